# Initial kernel scaffold; baseline (speedup 1.0000x reference)
import jax
import jax.numpy as jnp
from jax.experimental import pallas as pl


def kernel(x, edge_index, edge_index_neg, W1, b1, W2, b2, We1, be1, We2, be2):
    raise NotImplementedError("write your pallas kernel here")


# speedup vs baseline: 2.7133x; 2.7133x over previous
import functools

import jax
import jax.numpy as jnp
from jax import lax
from jax.experimental import pallas as pl
from jax.experimental.pallas import tpu as pltpu
from jax.experimental.pallas import tpu_sc as plsc

N = 10000
E = 320000
D = 128

NC, NS, L = 2, 16, 16
NW = NC * NS
ROWS_PER_W = (2 * E) // NW
CHUNK = 80
N_CHUNKS = ROWS_PER_W // CHUNK


def _sc_gather_mul(srcs, dsts, x):
    mesh = plsc.VectorSubcoreMesh(core_axis_name="c", subcore_axis_name="s")

    @functools.partial(
        pl.kernel,
        out_type=jax.ShapeDtypeStruct((2 * E, D), jnp.float32),
        mesh=mesh,
        scratch_types=[
            pltpu.VMEM((CHUNK,), jnp.int32),
            pltpu.VMEM((CHUNK,), jnp.int32),
            pltpu.VMEM((CHUNK, D), jnp.float32),
            pltpu.VMEM((CHUNK, D), jnp.float32),
            pltpu.VMEM((CHUNK, D), jnp.float32),
            pltpu.SemaphoreType.DMA,
            pltpu.SemaphoreType.DMA,
        ],
    )
    def k(srcs_hbm, dsts_hbm, x_hbm, h_hbm, idx_s, idx_d, a, b, o, sem0, sem1):
        wid = lax.axis_index("s") * NC + lax.axis_index("c")
        w_base = wid * ROWS_PER_W

        def chunk_body(kk, carry):
            base = w_base + kk * CHUNK
            cs = pltpu.async_copy(srcs_hbm.at[pl.ds(base, CHUNK)], idx_s, sem0)
            cd = pltpu.async_copy(dsts_hbm.at[pl.ds(base, CHUNK)], idx_d, sem1)
            cs.wait()
            cd.wait()
            ga = pltpu.async_copy(x_hbm.at[idx_s], a, sem0)
            gb = pltpu.async_copy(x_hbm.at[idx_d], b, sem1)
            ga.wait()
            gb.wait()

            def row_body(r, c2):
                for j in range(D // L):
                    sl = pl.ds(j * L, L)
                    o[r, sl] = a[r, sl] * b[r, sl]
                return c2

            lax.fori_loop(0, CHUNK, row_body, 0)
            pltpu.sync_copy(o, h_hbm.at[pl.ds(base, CHUNK)])
            return carry

        lax.fori_loop(0, N_CHUNKS, chunk_body, 0)

    return k(srcs, dsts, x)


B_TC = 2560
NBLK = E // B_TC


def _tc_body(hp_ref, hn_ref, w1_ref, b1_ref, w2_ref, b2_ref, o_ref):
    t = jnp.concatenate(
        [jnp.maximum(hp_ref[...], 0.0), jnp.maximum(hn_ref[...], 0.0)], axis=1)
    z = jnp.maximum(
        jnp.dot(t, w1_ref[...], preferred_element_type=jnp.float32)
        + b1_ref[...], 0.0)
    o_ref[...] = jax.nn.sigmoid(
        jnp.dot(z, w2_ref[...], preferred_element_type=jnp.float32)
        + b2_ref[...])


def _tc_mlp(h2, Wb1, bb1, Wb2, bb2):
    return pl.pallas_call(
        _tc_body,
        grid=(NBLK,),
        in_specs=[
            pl.BlockSpec((B_TC, D), lambda i: (i, 0)),
            pl.BlockSpec((B_TC, D), lambda i: (i + NBLK, 0)),
            pl.BlockSpec((2 * D, 3 * D), lambda i: (0, 0)),
            pl.BlockSpec((1, 3 * D), lambda i: (0, 0)),
            pl.BlockSpec((3 * D, 8), lambda i: (0, 0)),
            pl.BlockSpec((1, 8), lambda i: (0, 0)),
        ],
        out_specs=pl.BlockSpec((B_TC, 8), lambda i: (i, 0)),
        out_shape=jax.ShapeDtypeStruct((E, 8), jnp.float32),
    )(h2, h2, Wb1, bb1, Wb2, bb2)


@jax.jit
def kernel(x, edge_index, edge_index_neg, W1, b1, W2, b2, We1, be1, We2, be2):
    srcs = jnp.concatenate([edge_index[0], edge_index_neg[0]])
    dsts = jnp.concatenate([edge_index[1], edge_index_neg[1]])

    h2 = _sc_gather_mul(srcs, dsts, x)

    f32 = jnp.float32
    Wb1 = jnp.zeros((2 * D, 3 * D), f32)
    Wb1 = Wb1.at[:D, :D].set(W1).at[:D, D:2 * D].set(We1).at[D:, 2 * D:].set(We1)
    bb1 = jnp.concatenate([b1, be1, be1]).reshape(1, 3 * D)
    Wb2 = jnp.zeros((3 * D, 8), f32)
    Wb2 = Wb2.at[:D, :4].set(W2).at[D:2 * D, 4:5].set(We2).at[2 * D:, 5:6].set(We2)
    bb2 = jnp.concatenate([b2, be2, be2, jnp.zeros((2,), f32)]).reshape(1, 8)

    out8 = _tc_mlp(h2, Wb1, bb1, Wb2, bb2)
    return out8[:, :4], out8[:, 4], out8[:, 5]


# speedup vs baseline: 3.9092x; 1.4407x over previous
import functools

import jax
import jax.numpy as jnp
from jax import lax
from jax.experimental import pallas as pl
from jax.experimental.pallas import tpu as pltpu
from jax.experimental.pallas import tpu_sc as plsc

N = 10000
E = 320000
D = 128

NC, NS, L = 2, 16, 16
NW = NC * NS
ROWS_PER_W = (2 * E) // NW
CHUNK = 80
N_CHUNKS = ROWS_PER_W // CHUNK


def _sc_gather_mul(srcs, dsts, x):
    mesh = plsc.VectorSubcoreMesh(core_axis_name="c", subcore_axis_name="s")
    f32 = jnp.float32

    @functools.partial(
        pl.kernel,
        out_type=jax.ShapeDtypeStruct((2 * E, D), f32),
        mesh=mesh,
        scratch_types=[
            pltpu.VMEM((ROWS_PER_W,), jnp.int32),
            pltpu.VMEM((ROWS_PER_W,), jnp.int32),
            [pltpu.VMEM((CHUNK, D), f32)] * 2,
            [pltpu.VMEM((CHUNK, D), f32)] * 2,
            [pltpu.VMEM((CHUNK, D), f32)] * 2,
            [pltpu.SemaphoreType.DMA] * 2,
            [pltpu.SemaphoreType.DMA] * 2,
            pltpu.SemaphoreType.DMA,
        ],
    )
    def k(srcs_hbm, dsts_hbm, x_hbm, h_hbm, idx_s, idx_d, a, b, o,
          sem_g, sem_w, sem_i):
        wid = lax.axis_index("s") * NC + lax.axis_index("c")
        w_base = wid * ROWS_PER_W

        ci = pltpu.async_copy(srcs_hbm.at[pl.ds(w_base, ROWS_PER_W)], idx_s,
                              sem_i)
        cd = pltpu.async_copy(dsts_hbm.at[pl.ds(w_base, ROWS_PER_W)], idx_d,
                              sem_i)
        ci.wait()
        cd.wait()

        def fire(kk, p):
            pltpu.async_copy(x_hbm.at[idx_s.at[pl.ds(kk * CHUNK, CHUNK)]],
                             a[p], sem_g[p])
            pltpu.async_copy(x_hbm.at[idx_d.at[pl.ds(kk * CHUNK, CHUNK)]],
                             b[p], sem_g[p])

        fire(0, 0)

        @pl.loop(0, N_CHUNKS, step=2)
        def chunk_pair(k0):
            for p in range(2):
                kk = k0 + p

                @pl.when(kk + 1 < N_CHUNKS)
                def _():
                    fire(kk + 1, 1 - p)

                pltpu.make_async_copy(x_hbm.at[idx_s.at[pl.ds(0, CHUNK)]],
                                      a[p], sem_g[p]).wait()
                pltpu.make_async_copy(x_hbm.at[idx_d.at[pl.ds(0, CHUNK)]],
                                      b[p], sem_g[p]).wait()

                @pl.when(kk >= 2)
                def _():
                    pltpu.make_async_copy(
                        o[p], h_hbm.at[pl.ds(w_base, CHUNK)],
                        sem_w[p]).wait()

                @plsc.parallel_loop(0, CHUNK, 1, unroll=4)
                def row_body(r):
                    for j in range(D // L):
                        sl = pl.ds(j * L, L)
                        o[p][r, sl] = a[p][r, sl] * b[p][r, sl]

                pltpu.async_copy(
                    o[p], h_hbm.at[pl.ds(w_base + kk * CHUNK, CHUNK)],
                    sem_w[p])

        for p in range(2):
            pltpu.make_async_copy(o[p], h_hbm.at[pl.ds(w_base, CHUNK)],
                                  sem_w[p]).wait()

    return k(srcs, dsts, x)


B_TC = 2560
NBLK = E // B_TC


def _tc_body(hp_ref, hn_ref, w1_ref, b1_ref, w2_ref, b2_ref, o_ref):
    t = jnp.concatenate(
        [jnp.maximum(hp_ref[...], 0.0), jnp.maximum(hn_ref[...], 0.0)], axis=1)
    z = jnp.maximum(
        jnp.dot(t, w1_ref[...], preferred_element_type=jnp.float32)
        + b1_ref[...], 0.0)
    o_ref[...] = jax.nn.sigmoid(
        jnp.dot(z, w2_ref[...], preferred_element_type=jnp.float32)
        + b2_ref[...])


def _tc_mlp(h2, Wb1, bb1, Wb2, bb2):
    return pl.pallas_call(
        _tc_body,
        grid=(NBLK,),
        in_specs=[
            pl.BlockSpec((B_TC, D), lambda i: (i, 0)),
            pl.BlockSpec((B_TC, D), lambda i: (i + NBLK, 0)),
            pl.BlockSpec((2 * D, 3 * D), lambda i: (0, 0)),
            pl.BlockSpec((1, 3 * D), lambda i: (0, 0)),
            pl.BlockSpec((3 * D, 8), lambda i: (0, 0)),
            pl.BlockSpec((1, 8), lambda i: (0, 0)),
        ],
        out_specs=pl.BlockSpec((B_TC, 8), lambda i: (i, 0)),
        out_shape=jax.ShapeDtypeStruct((E, 8), jnp.float32),
    )(h2, h2, Wb1, bb1, Wb2, bb2)


@jax.jit
def kernel(x, edge_index, edge_index_neg, W1, b1, W2, b2, We1, be1, We2, be2):
    srcs = jnp.concatenate([edge_index[0], edge_index_neg[0]])
    dsts = jnp.concatenate([edge_index[1], edge_index_neg[1]])

    h2 = _sc_gather_mul(srcs, dsts, x)

    f32 = jnp.float32
    Wb1 = jnp.zeros((2 * D, 3 * D), f32)
    Wb1 = Wb1.at[:D, :D].set(W1).at[:D, D:2 * D].set(We1).at[D:, 2 * D:].set(We1)
    bb1 = jnp.concatenate([b1, be1, be1]).reshape(1, 3 * D)
    Wb2 = jnp.zeros((3 * D, 8), f32)
    Wb2 = Wb2.at[:D, :4].set(W2).at[D:2 * D, 4:5].set(We2).at[2 * D:, 5:6].set(We2)
    bb2 = jnp.concatenate([b2, be2, be2, jnp.zeros((2,), f32)]).reshape(1, 8)

    out8 = _tc_mlp(h2, Wb1, bb1, Wb2, bb2)
    return out8[:, :4], out8[:, 4], out8[:, 5]
